# baseline (speedup 1.0000x reference)
import functools

import jax
import jax.numpy as jnp
from jax import lax
from jax.experimental import pallas as pl
from jax.experimental.pallas import tpu as pltpu
from jax.experimental.pallas import tpu_sc as plsc

NC = 2
NS = 16
NW = NC * NS
SPLIT = 5


def _pick_ch(epw):
    for c in range(128, 7, -8):
        if epw % c == 0:
            return c
    raise ValueError(epw)


def _sc_mesh():
    return plsc.VectorSubcoreMesh(
        core_axis_name="c", subcore_axis_name="s", num_cores=NC, num_subcores=NS
    )


def _pipeline(nchunk, proc, pref):
    npairs = (nchunk - 2) // 2
    pref(0, 0)
    pref(1, 1)

    def body(j2, carry):
        for b in range(2):
            jj = 2 * j2 + b
            proc(jj, b)
            pref(jj + 2, b)
        return carry

    lax.fori_loop(0, npairs, body, 0)
    done = 2 * npairs
    if nchunk % 2 == 1:
        proc(done, done % 2)
        pref(nchunk - 1, (nchunk - 1) % 2)
        done += 1
    for jj in range(done, nchunk):
        proc(jj, jj % 2)


def _make_gather(E, N, DN):
    epw = E // NW
    ch = _pick_ch(epw)
    nchunk = epw // ch
    DH = DN // 2

    @functools.partial(
        pl.kernel,
        out_type=(
            jax.ShapeDtypeStruct((E, DH), jnp.int32),
            jax.ShapeDtypeStruct((E, DH), jnp.int32),
        ),
        mesh=_sc_mesh(),
        scratch_types=[
            pltpu.VMEM((epw,), jnp.int32),
            pltpu.VMEM((epw,), jnp.int32),
            pltpu.VMEM((2, ch, DN), jnp.float32),
            pltpu.VMEM((2, ch, DN), jnp.float32),
            pltpu.VMEM((2, ch, DH), jnp.int32),
            pltpu.VMEM((2, ch, DH), jnp.int32),
            pltpu.SemaphoreType.DMA((2,)),
            pltpu.SemaphoreType.DMA((2,)),
        ],
    )
    def gather_k(x_hbm, row_hbm, col_hbm, src_hbm, dest_hbm,
                 idx_r, idx_c, buf_r, buf_c, pk_r, pk_c, sem_r, sem_c):
        wid = lax.axis_index("s") * NC + lax.axis_index("c")
        base0 = wid * epw
        pltpu.sync_copy(row_hbm.at[pl.ds(base0, epw)], idx_r)
        pltpu.sync_copy(col_hbm.at[pl.ds(base0, epw)], idx_c)

        def pref(jj, b):
            pltpu.async_copy(
                x_hbm.at[idx_r.at[pl.ds(jj * ch, ch)]], buf_r.at[b], sem_r.at[b])
            pltpu.async_copy(
                x_hbm.at[idx_c.at[pl.ds(jj * ch, ch)]], buf_c.at[b], sem_c.at[b])

        def pack_rows(buf, pk, b):
            def rowfn(i, carry):
                for g in range(0, DH, 16):
                    lo = jax.lax.bitcast_convert_type(
                        buf[b, i, pl.ds(g, 16)], jnp.int32)
                    hi = jax.lax.bitcast_convert_type(
                        buf[b, i, pl.ds(DH + g, 16)], jnp.int32)
                    pk[b, i, pl.ds(g, 16)] = (
                        (hi & jnp.int32(-65536))
                        | ((lo >> 16) & jnp.int32(65535)))
                return carry

            lax.fori_loop(0, ch, rowfn, 0)

        def proc(jj, b):
            base = base0 + jj * ch
            pltpu.make_async_copy(
                x_hbm.at[idx_r.at[pl.ds(jj * ch, ch)]], buf_r.at[b], sem_r.at[b]
            ).wait()
            pack_rows(buf_r, pk_r, b)
            pltpu.make_async_copy(
                x_hbm.at[idx_c.at[pl.ds(jj * ch, ch)]], buf_c.at[b], sem_c.at[b]
            ).wait()
            pack_rows(buf_c, pk_c, b)
            pltpu.sync_copy(pk_r.at[b], src_hbm.at[pl.ds(base, ch)])
            pltpu.sync_copy(pk_c.at[b], dest_hbm.at[pl.ds(base, ch)])

        _pipeline(nchunk, proc, pref)

    return gather_k


def _make_scatter(E, N, W, WC):
    epw = E // NS
    ch = _pick_ch(epw)
    nchunk = epw // ch
    npad = ((N + NS * 8 - 1) // (NS * 8)) * (NS * 8)
    rows_per_sub = npad // NS

    @functools.partial(
        pl.kernel,
        out_type=jax.ShapeDtypeStruct((NC, npad, W), jnp.float32),
        mesh=_sc_mesh(),
        scratch_types=[
            pltpu.VMEM((epw,), jnp.int32),
            pltpu.VMEM((2, ch, W), jnp.float32),
            pltpu.VMEM_SHARED((npad, W), jnp.float32),
            pltpu.SemaphoreType.DMA((2,)),
        ],
    )
    def scatter_k(v0_hbm, v1_hbm, col_hbm, z_hbm, out_hbm,
                  idx_v, val_v, acc_sh, sem):
        c = lax.axis_index("c")
        s = lax.axis_index("s")
        pltpu.sync_copy(z_hbm, acc_sh.at[pl.ds(s * rows_per_sub, rows_per_sub)])
        base0 = s * epw
        pltpu.sync_copy(col_hbm.at[pl.ds(base0, epw)], idx_v)
        plsc.subcore_barrier()

        def scat(jj, b):
            pltpu.sync_copy(
                val_v.at[b], acc_sh.at[idx_v.at[pl.ds(jj * ch, ch)]], add=True)

        def make(v_hbm):
            def pref(jj, b):
                pltpu.async_copy(
                    v_hbm.at[pl.ds(base0 + jj * ch, ch)], val_v.at[b], sem.at[b])

            def proc(jj, b):
                pltpu.make_async_copy(
                    v_hbm.at[pl.ds(base0, ch)], val_v.at[b], sem.at[b]).wait()
                scat(jj, b)

            return proc, pref

        @pl.when(c == 0)
        def _():
            _pipeline(nchunk, *make(v0_hbm))

        @pl.when(c == 1)
        def _():
            _pipeline(nchunk, *make(v1_hbm))

        plsc.subcore_barrier()
        pltpu.sync_copy(
            acc_sh.at[pl.ds(s * rows_per_sub, rows_per_sub)],
            out_hbm.at[c, pl.ds(s * rows_per_sub, rows_per_sub)],
        )

    return scatter_k


def _unpack_bf16(word):
    lo = jax.lax.bitcast_convert_type(
        jax.lax.shift_left(word, 16), jnp.float32).astype(jnp.bfloat16)
    hi = jax.lax.bitcast_convert_type(
        jax.lax.bitwise_and(word, jnp.int32(-65536)), jnp.float32
    ).astype(jnp.bfloat16)
    return lo, hi


def _edge_body(src_ref, dest_ref, ea_ref, W1e_ref, b1e_ref, W2e_ref, b2e_ref,
               W1n_ref, b1n_ref, W2n_ref, b2n_ref, edge_ref, v0_ref, v1_ref):
    slo, shi = _unpack_bf16(src_ref[...])
    dlo, dhi = _unpack_bf16(dest_ref[...])
    ea = ea_ref[...].astype(jnp.bfloat16)
    H = slo.shape[1]
    W1e = W1e_ref[...].astype(jnp.bfloat16)
    he = (
        jnp.dot(slo, W1e[:H], preferred_element_type=jnp.float32)
        + jnp.dot(shi, W1e[H:2 * H], preferred_element_type=jnp.float32)
        + jnp.dot(dlo, W1e[2 * H:3 * H], preferred_element_type=jnp.float32)
        + jnp.dot(dhi, W1e[3 * H:4 * H], preferred_element_type=jnp.float32)
        + jnp.dot(ea, W1e[4 * H:], preferred_element_type=jnp.float32)
        + b1e_ref[...]
    )
    he = jnp.maximum(he, 0.0).astype(jnp.bfloat16)
    edge_ref[...] = (
        jnp.dot(he, W2e_ref[...].astype(jnp.bfloat16),
                preferred_element_type=jnp.float32) + b2e_ref[...]
    )
    W1n = W1n_ref[...].astype(jnp.bfloat16)
    hn = (
        jnp.dot(slo, W1n[:H], preferred_element_type=jnp.float32)
        + jnp.dot(shi, W1n[H:2 * H], preferred_element_type=jnp.float32)
        + jnp.dot(ea, W1n[2 * H:], preferred_element_type=jnp.float32)
        + b1n_ref[...]
    )
    hn = jnp.maximum(hn, 0.0).astype(jnp.bfloat16)
    r = jnp.dot(hn, W2n_ref[...].astype(jnp.bfloat16),
                preferred_element_type=jnp.float32) + b2n_ref[...]
    bk = r.shape[0]
    lanes = v0_ref.shape[1]
    v0_ref[...] = r[:, :lanes]
    pad = 2 * lanes - r.shape[1] - 1
    v1_ref[...] = jnp.concatenate(
        [r[:, lanes:], jnp.ones((bk, 1), jnp.float32),
         jnp.zeros((bk, pad), jnp.float32)],
        axis=1,
    )


def _node_body(x_ref, *args):
    S_refs = args[:-5]
    W1m_ref, b1m_ref, W2m_ref, b2m_ref, node_ref = args[-5:]
    x = x_ref[...]
    DN = x.shape[1]
    DM = W1m_ref.shape[0] - DN
    lanes = S_refs[0].shape[2]
    S0 = sum(Sr[0] for Sr in S_refs)
    S1 = sum(Sr[1] for Sr in S_refs)
    summed = jnp.concatenate([S0, S1[:, :DM - lanes]], axis=1)
    cnt = S1[:, DM - lanes:DM - lanes + 1]
    mean = summed / jnp.maximum(cnt, 1.0)
    W1m = W1m_ref[...]
    h = (
        jnp.dot(x, W1m[:DN], preferred_element_type=jnp.float32)
        + jnp.dot(mean, W1m[DN:], preferred_element_type=jnp.float32)
        + b1m_ref[...]
    )
    h = jnp.maximum(h, 0.0)
    node_ref[...] = (
        jnp.dot(h, W2m_ref[...], preferred_element_type=jnp.float32) + b2m_ref[...]
    )


def kernel(x, edge_index, edge_attr,
           W1e, b1e, W2e, b2e,
           W1n, b1n, W2n, b2n,
           W1m, b1m, W2m, b2m):
    N, DN = x.shape
    E = edge_index.shape[1]
    DE = edge_attr.shape[1]
    DM = W2n.shape[1]
    LANES = 128
    WC = DM - LANES
    NL = W2m.shape[1]
    EH = E // SPLIT

    full = lambda shape: pl.BlockSpec(shape, lambda i: (0,) * len(shape))
    npad = ((N + NS * 8 - 1) // (NS * 8)) * (NS * 8)
    zeros_tile = jnp.zeros((npad // NS, LANES), jnp.float32)

    gather_h = _make_gather(EH, N, DN)
    scatter_h = _make_scatter(EH, N, LANES, WC)

    def edge_tc(srch, desth, eah):
        BK = 2000
        return pl.pallas_call(
            _edge_body,
            grid=(EH // BK,),
            in_specs=[
                pl.BlockSpec((BK, DN // 2), lambda i: (i, 0)),
                pl.BlockSpec((BK, DN // 2), lambda i: (i, 0)),
                pl.BlockSpec((BK, DE), lambda i: (i, 0)),
                full(W1e.shape), full((1, b1e.shape[0])),
                full(W2e.shape), full((1, b2e.shape[0])),
                full(W1n.shape), full((1, b1n.shape[0])),
                full(W2n.shape), full((1, b2n.shape[0])),
            ],
            out_specs=[
                pl.BlockSpec((BK, W2e.shape[1]), lambda i: (i, 0)),
                pl.BlockSpec((BK, LANES), lambda i: (i, 0)),
                pl.BlockSpec((BK, LANES), lambda i: (i, 0)),
            ],
            out_shape=[
                jax.ShapeDtypeStruct((EH, W2e.shape[1]), jnp.float32),
                jax.ShapeDtypeStruct((EH, LANES), jnp.float32),
                jax.ShapeDtypeStruct((EH, LANES), jnp.float32),
            ],
        )(srch, desth, eah,
          W1e, b1e.reshape(1, -1), W2e, b2e.reshape(1, -1),
          W1n, b1n.reshape(1, -1), W2n, b2n.reshape(1, -1))

    edges = []
    S = []
    for h in range(SPLIT):
        rowh = lax.slice_in_dim(edge_index[0], h * EH, (h + 1) * EH)
        colh = lax.slice_in_dim(edge_index[1], h * EH, (h + 1) * EH)
        eah = lax.slice_in_dim(edge_attr, h * EH, (h + 1) * EH)
        srch, desth = gather_h(x, rowh, colh)
        edge_h, v0_h, v1_h = edge_tc(srch, desth, eah)
        S.append(scatter_h(v0_h, v1_h, colh, zeros_tile))
        edges.append(edge_h)

    edge_out = jnp.concatenate(edges, axis=0)

    BN = 2000
    node = pl.pallas_call(
        _node_body,
        grid=(N // BN,),
        in_specs=[
            pl.BlockSpec((BN, DN), lambda i: (i, 0)),
        ] + [
            pl.BlockSpec((NC, BN, LANES), lambda i: (0, i, 0))
            for _ in range(SPLIT)
        ] + [
            full(W1m.shape), full((1, b1m.shape[0])),
            full(W2m.shape), full((1, b2m.shape[0])),
        ],
        out_specs=pl.BlockSpec((BN, NL), lambda i: (i, 0)),
        out_shape=jax.ShapeDtypeStruct((N, NL), jnp.float32),
    )(x, *S, W1m, b1m.reshape(1, -1), W2m, b2m.reshape(1, -1))

    return (node, edge_out)


# speedup vs baseline: 1.2471x; 1.2471x over previous
import functools

import jax
import jax.numpy as jnp
from jax import lax
from jax.experimental import pallas as pl
from jax.experimental.pallas import tpu as pltpu
from jax.experimental.pallas import tpu_sc as plsc

NC = 2
NS = 16
NW = NC * NS
SPLIT = 5


def _pick_ch(epw):
    for c in range(128, 7, -8):
        if epw % c == 0:
            return c
    raise ValueError(epw)


def _sc_mesh():
    return plsc.VectorSubcoreMesh(
        core_axis_name="c", subcore_axis_name="s", num_cores=NC, num_subcores=NS
    )


def _pipeline(nchunk, proc, pref):
    npairs = (nchunk - 2) // 2
    pref(0, 0)
    pref(1, 1)

    def body(j2, carry):
        for b in range(2):
            jj = 2 * j2 + b
            proc(jj, b)
            pref(jj + 2, b)
        return carry

    lax.fori_loop(0, npairs, body, 0)
    done = 2 * npairs
    if nchunk % 2 == 1:
        proc(done, done % 2)
        pref(nchunk - 1, (nchunk - 1) % 2)
        done += 1
    for jj in range(done, nchunk):
        proc(jj, jj % 2)


def _make_gather(E, N, DN, dtype):
    epw = E // NW
    ch = _pick_ch(epw)
    nchunk = epw // ch

    @functools.partial(
        pl.kernel,
        out_type=(
            jax.ShapeDtypeStruct((E, DN), dtype),
            jax.ShapeDtypeStruct((E, DN), dtype),
        ),
        mesh=_sc_mesh(),
        scratch_types=[
            pltpu.VMEM((epw,), jnp.int32),
            pltpu.VMEM((epw,), jnp.int32),
            pltpu.VMEM((2, ch, DN), dtype),
            pltpu.VMEM((2, ch, DN), dtype),
            pltpu.SemaphoreType.DMA((2,)),
            pltpu.SemaphoreType.DMA((2,)),
        ],
    )
    def gather_k(x_hbm, row_hbm, col_hbm, src_hbm, dest_hbm,
                 idx_r, idx_c, buf_r, buf_c, sem_r, sem_c):
        wid = lax.axis_index("s") * NC + lax.axis_index("c")
        base0 = wid * epw
        pltpu.sync_copy(row_hbm.at[pl.ds(base0, epw)], idx_r)
        pltpu.sync_copy(col_hbm.at[pl.ds(base0, epw)], idx_c)

        def pref(jj, b):
            pltpu.async_copy(
                x_hbm.at[idx_r.at[pl.ds(jj * ch, ch)]], buf_r.at[b], sem_r.at[b])
            pltpu.async_copy(
                x_hbm.at[idx_c.at[pl.ds(jj * ch, ch)]], buf_c.at[b], sem_c.at[b])

        def proc(jj, b):
            base = base0 + jj * ch
            pltpu.make_async_copy(
                x_hbm.at[idx_r.at[pl.ds(jj * ch, ch)]], buf_r.at[b], sem_r.at[b]
            ).wait()
            pltpu.make_async_copy(
                x_hbm.at[idx_c.at[pl.ds(jj * ch, ch)]], buf_c.at[b], sem_c.at[b]
            ).wait()
            pltpu.sync_copy(buf_r.at[b], src_hbm.at[pl.ds(base, ch)])
            pltpu.sync_copy(buf_c.at[b], dest_hbm.at[pl.ds(base, ch)])

        _pipeline(nchunk, proc, pref)

    return gather_k


def _make_scatter(E, N, W, WC):
    epw = E // NS
    ch = _pick_ch(epw)
    nchunk = epw // ch
    npad = ((N + NS * 8 - 1) // (NS * 8)) * (NS * 8)
    rows_per_sub = npad // NS

    @functools.partial(
        pl.kernel,
        out_type=jax.ShapeDtypeStruct((NC, npad, W), jnp.float32),
        mesh=_sc_mesh(),
        scratch_types=[
            pltpu.VMEM((epw,), jnp.int32),
            pltpu.VMEM((2, ch, W), jnp.float32),
            pltpu.VMEM_SHARED((npad, W), jnp.float32),
            pltpu.SemaphoreType.DMA((2,)),
        ],
    )
    def scatter_k(v0_hbm, v1_hbm, col_hbm, z_hbm, out_hbm,
                  idx_v, val_v, acc_sh, sem):
        c = lax.axis_index("c")
        s = lax.axis_index("s")
        pltpu.sync_copy(z_hbm, acc_sh.at[pl.ds(s * rows_per_sub, rows_per_sub)])
        base0 = s * epw
        pltpu.sync_copy(col_hbm.at[pl.ds(base0, epw)], idx_v)
        plsc.subcore_barrier()

        def scat(jj, b):
            pltpu.sync_copy(
                val_v.at[b], acc_sh.at[idx_v.at[pl.ds(jj * ch, ch)]], add=True)

        def make(v_hbm):
            def pref(jj, b):
                pltpu.async_copy(
                    v_hbm.at[pl.ds(base0 + jj * ch, ch)], val_v.at[b], sem.at[b])

            def proc(jj, b):
                pltpu.make_async_copy(
                    v_hbm.at[pl.ds(base0, ch)], val_v.at[b], sem.at[b]).wait()
                scat(jj, b)

            return proc, pref

        @pl.when(c == 0)
        def _():
            _pipeline(nchunk, *make(v0_hbm))

        @pl.when(c == 1)
        def _():
            _pipeline(nchunk, *make(v1_hbm))

        plsc.subcore_barrier()
        pltpu.sync_copy(
            acc_sh.at[pl.ds(s * rows_per_sub, rows_per_sub)],
            out_hbm.at[c, pl.ds(s * rows_per_sub, rows_per_sub)],
        )

    return scatter_k


def _edge_body(src_ref, dest_ref, ea_ref, W1e_ref, b1e_ref, W2e_ref, b2e_ref,
               W1n_ref, b1n_ref, W2n_ref, b2n_ref, edge_ref, v0_ref, v1_ref):
    src = src_ref[...].astype(jnp.bfloat16)
    dest = dest_ref[...].astype(jnp.bfloat16)
    ea = ea_ref[...].astype(jnp.bfloat16)
    DN = src.shape[1]
    W1e = W1e_ref[...].astype(jnp.bfloat16)
    he = (
        jnp.dot(src, W1e[:DN], preferred_element_type=jnp.float32)
        + jnp.dot(dest, W1e[DN:2 * DN], preferred_element_type=jnp.float32)
        + jnp.dot(ea, W1e[2 * DN:], preferred_element_type=jnp.float32)
        + b1e_ref[...]
    )
    he = jnp.maximum(he, 0.0).astype(jnp.bfloat16)
    edge_ref[...] = (
        jnp.dot(he, W2e_ref[...].astype(jnp.bfloat16),
                preferred_element_type=jnp.float32) + b2e_ref[...]
    )
    W1n = W1n_ref[...].astype(jnp.bfloat16)
    hn = (
        jnp.dot(src, W1n[:DN], preferred_element_type=jnp.float32)
        + jnp.dot(ea, W1n[DN:], preferred_element_type=jnp.float32)
        + b1n_ref[...]
    )
    hn = jnp.maximum(hn, 0.0).astype(jnp.bfloat16)
    r = jnp.dot(hn, W2n_ref[...].astype(jnp.bfloat16),
                preferred_element_type=jnp.float32) + b2n_ref[...]
    bk = r.shape[0]
    lanes = v0_ref.shape[1]
    v0_ref[...] = r[:, :lanes]
    pad = 2 * lanes - r.shape[1] - 1
    v1_ref[...] = jnp.concatenate(
        [r[:, lanes:], jnp.ones((bk, 1), jnp.float32),
         jnp.zeros((bk, pad), jnp.float32)],
        axis=1,
    )


def _node_body(x_ref, *args):
    S_refs = args[:-5]
    W1m_ref, b1m_ref, W2m_ref, b2m_ref, node_ref = args[-5:]
    x = x_ref[...]
    DN = x.shape[1]
    DM = W1m_ref.shape[0] - DN
    lanes = S_refs[0].shape[2]
    S0 = sum(Sr[0] for Sr in S_refs)
    S1 = sum(Sr[1] for Sr in S_refs)
    summed = jnp.concatenate([S0, S1[:, :DM - lanes]], axis=1)
    cnt = S1[:, DM - lanes:DM - lanes + 1]
    mean = summed / jnp.maximum(cnt, 1.0)
    W1m = W1m_ref[...]
    h = (
        jnp.dot(x, W1m[:DN], preferred_element_type=jnp.float32)
        + jnp.dot(mean, W1m[DN:], preferred_element_type=jnp.float32)
        + b1m_ref[...]
    )
    h = jnp.maximum(h, 0.0)
    node_ref[...] = (
        jnp.dot(h, W2m_ref[...], preferred_element_type=jnp.float32) + b2m_ref[...]
    )


def kernel(x, edge_index, edge_attr,
           W1e, b1e, W2e, b2e,
           W1n, b1n, W2n, b2n,
           W1m, b1m, W2m, b2m):
    N, DN = x.shape
    E = edge_index.shape[1]
    DE = edge_attr.shape[1]
    DM = W2n.shape[1]
    LANES = 128
    WC = DM - LANES
    NL = W2m.shape[1]
    EH = E // SPLIT

    full = lambda shape: pl.BlockSpec(shape, lambda i: (0,) * len(shape))
    npad = ((N + NS * 8 - 1) // (NS * 8)) * (NS * 8)
    zeros_tile = jnp.zeros((npad // NS, LANES), jnp.float32)

    gather_h = _make_gather(EH, N, DN, jnp.float32)
    scatter_h = _make_scatter(EH, N, LANES, WC)

    def edge_tc(srch, desth, eah):
        BK = 2000
        return pl.pallas_call(
            _edge_body,
            grid=(EH // BK,),
            in_specs=[
                pl.BlockSpec((BK, DN), lambda i: (i, 0)),
                pl.BlockSpec((BK, DN), lambda i: (i, 0)),
                pl.BlockSpec((BK, DE), lambda i: (i, 0)),
                full(W1e.shape), full((1, b1e.shape[0])),
                full(W2e.shape), full((1, b2e.shape[0])),
                full(W1n.shape), full((1, b1n.shape[0])),
                full(W2n.shape), full((1, b2n.shape[0])),
            ],
            out_specs=[
                pl.BlockSpec((BK, W2e.shape[1]), lambda i: (i, 0)),
                pl.BlockSpec((BK, LANES), lambda i: (i, 0)),
                pl.BlockSpec((BK, LANES), lambda i: (i, 0)),
            ],
            out_shape=[
                jax.ShapeDtypeStruct((EH, W2e.shape[1]), jnp.float32),
                jax.ShapeDtypeStruct((EH, LANES), jnp.float32),
                jax.ShapeDtypeStruct((EH, LANES), jnp.float32),
            ],
        )(srch, desth, eah,
          W1e, b1e.reshape(1, -1), W2e, b2e.reshape(1, -1),
          W1n, b1n.reshape(1, -1), W2n, b2n.reshape(1, -1))

    edges = []
    S = []
    for h in range(SPLIT):
        rowh = lax.slice_in_dim(edge_index[0], h * EH, (h + 1) * EH)
        colh = lax.slice_in_dim(edge_index[1], h * EH, (h + 1) * EH)
        eah = lax.slice_in_dim(edge_attr, h * EH, (h + 1) * EH)
        srch, desth = gather_h(x, rowh, colh)
        edge_h, v0_h, v1_h = edge_tc(srch, desth, eah)
        S.append(scatter_h(v0_h, v1_h, colh, zeros_tile))
        edges.append(edge_h)

    edge_out = jnp.concatenate(edges, axis=0)

    BN = 2000
    node = pl.pallas_call(
        _node_body,
        grid=(N // BN,),
        in_specs=[
            pl.BlockSpec((BN, DN), lambda i: (i, 0)),
        ] + [
            pl.BlockSpec((NC, BN, LANES), lambda i: (0, i, 0))
            for _ in range(SPLIT)
        ] + [
            full(W1m.shape), full((1, b1m.shape[0])),
            full(W2m.shape), full((1, b2m.shape[0])),
        ],
        out_specs=pl.BlockSpec((BN, NL), lambda i: (i, 0)),
        out_shape=jax.ShapeDtypeStruct((N, NL), jnp.float32),
    )(x, *S, W1m, b1m.reshape(1, -1), W2m, b2m.reshape(1, -1))

    return (node, edge_out)


# speedup vs baseline: 1.2762x; 1.0233x over previous
import functools

import jax
import jax.numpy as jnp
from jax import lax
from jax.experimental import pallas as pl
from jax.experimental.pallas import tpu as pltpu
from jax.experimental.pallas import tpu_sc as plsc

NC = 2
NS = 16
NW = NC * NS
SPLIT = 5


def _pick_ch(epw):
    for c in range(128, 7, -8):
        if epw % c == 0:
            return c
    raise ValueError(epw)


def _sc_mesh():
    return plsc.VectorSubcoreMesh(
        core_axis_name="c", subcore_axis_name="s", num_cores=NC, num_subcores=NS
    )


def _pipeline(nchunk, proc, pref):
    npairs = (nchunk - 2) // 2
    pref(0, 0)
    pref(1, 1)

    def body(j2, carry):
        for b in range(2):
            jj = 2 * j2 + b
            proc(jj, b)
            pref(jj + 2, b)
        return carry

    lax.fori_loop(0, npairs, body, 0)
    done = 2 * npairs
    if nchunk % 2 == 1:
        proc(done, done % 2)
        pref(nchunk - 1, (nchunk - 1) % 2)
        done += 1
    for jj in range(done, nchunk):
        proc(jj, jj % 2)


def _make_gather(E, N, DN, dtype):
    epw = E // NW
    ch = _pick_ch(epw)
    nchunk = epw // ch

    @functools.partial(
        pl.kernel,
        out_type=(
            jax.ShapeDtypeStruct((E, DN), dtype),
            jax.ShapeDtypeStruct((E, DN), dtype),
        ),
        mesh=_sc_mesh(),
        scratch_types=[
            pltpu.VMEM((epw,), jnp.int32),
            pltpu.VMEM((epw,), jnp.int32),
            pltpu.VMEM((2, ch, DN), dtype),
            pltpu.VMEM((2, ch, DN), dtype),
            pltpu.SemaphoreType.DMA((2,)),
            pltpu.SemaphoreType.DMA((2,)),
        ],
    )
    def gather_k(x_hbm, row_hbm, col_hbm, src_hbm, dest_hbm,
                 idx_r, idx_c, buf_r, buf_c, sem_r, sem_c):
        wid = lax.axis_index("s") * NC + lax.axis_index("c")
        base0 = wid * epw
        pltpu.sync_copy(row_hbm.at[pl.ds(base0, epw)], idx_r)
        pltpu.sync_copy(col_hbm.at[pl.ds(base0, epw)], idx_c)

        def pref(jj, b):
            pltpu.async_copy(
                x_hbm.at[idx_r.at[pl.ds(jj * ch, ch)]], buf_r.at[b], sem_r.at[b])
            pltpu.async_copy(
                x_hbm.at[idx_c.at[pl.ds(jj * ch, ch)]], buf_c.at[b], sem_c.at[b])

        def proc(jj, b):
            base = base0 + jj * ch
            pltpu.make_async_copy(
                x_hbm.at[idx_r.at[pl.ds(jj * ch, ch)]], buf_r.at[b], sem_r.at[b]
            ).wait()
            pltpu.make_async_copy(
                x_hbm.at[idx_c.at[pl.ds(jj * ch, ch)]], buf_c.at[b], sem_c.at[b]
            ).wait()
            pltpu.sync_copy(buf_r.at[b], src_hbm.at[pl.ds(base, ch)])
            pltpu.sync_copy(buf_c.at[b], dest_hbm.at[pl.ds(base, ch)])

        _pipeline(nchunk, proc, pref)

    return gather_k


def _make_scatter(E, N, W, WC):
    epw = E // NS
    ch = _pick_ch(epw)
    nchunk = epw // ch
    npad = ((N + NS * 8 - 1) // (NS * 8)) * (NS * 8)
    rows_per_sub = npad // NS

    @functools.partial(
        pl.kernel,
        out_type=jax.ShapeDtypeStruct((NC, npad, W), jnp.float32),
        mesh=_sc_mesh(),
        scratch_types=[
            pltpu.VMEM((epw,), jnp.int32),
            pltpu.VMEM((2, ch, W), jnp.float32),
            pltpu.VMEM_SHARED((npad, W), jnp.float32),
            pltpu.SemaphoreType.DMA((2,)),
        ],
    )
    def scatter_k(v0_hbm, v1_hbm, col_hbm, z_hbm, out_hbm,
                  idx_v, val_v, acc_sh, sem):
        c = lax.axis_index("c")
        s = lax.axis_index("s")
        pltpu.sync_copy(z_hbm, acc_sh.at[pl.ds(s * rows_per_sub, rows_per_sub)])
        base0 = s * epw
        pltpu.sync_copy(col_hbm.at[pl.ds(base0, epw)], idx_v)
        plsc.subcore_barrier()

        def scat(jj, b):
            pltpu.sync_copy(
                val_v.at[b], acc_sh.at[idx_v.at[pl.ds(jj * ch, ch)]], add=True)

        def make(v_hbm):
            def pref(jj, b):
                pltpu.async_copy(
                    v_hbm.at[pl.ds(base0 + jj * ch, ch)], val_v.at[b], sem.at[b])

            def proc(jj, b):
                pltpu.make_async_copy(
                    v_hbm.at[pl.ds(base0, ch)], val_v.at[b], sem.at[b]).wait()
                scat(jj, b)

            return proc, pref

        @pl.when(c == 0)
        def _():
            _pipeline(nchunk, *make(v0_hbm))

        @pl.when(c == 1)
        def _():
            _pipeline(nchunk, *make(v1_hbm))

        plsc.subcore_barrier()
        pltpu.sync_copy(
            acc_sh.at[pl.ds(s * rows_per_sub, rows_per_sub)],
            out_hbm.at[c, pl.ds(s * rows_per_sub, rows_per_sub)],
        )

    return scatter_k


def _edge_body(src_ref, dest_ref, ea_ref, W1e_ref, b1e_ref, W2e_ref, b2e_ref,
               W1n_ref, b1n_ref, W2n_ref, b2n_ref, edge_ref, v0_ref, v1_ref):
    src = src_ref[...].astype(jnp.bfloat16)
    dest = dest_ref[...].astype(jnp.bfloat16)
    ea = ea_ref[...].astype(jnp.bfloat16)
    DN = src.shape[1]
    sd = jnp.concatenate([src, dest], axis=1)
    W1e = W1e_ref[...].astype(jnp.bfloat16)
    he = (
        jnp.dot(sd, W1e[:2 * DN], preferred_element_type=jnp.float32)
        + jnp.dot(ea, W1e[2 * DN:], preferred_element_type=jnp.float32)
        + b1e_ref[...]
    )
    he = jnp.maximum(he, 0.0).astype(jnp.bfloat16)
    edge_ref[...] = (
        jnp.dot(he, W2e_ref[...].astype(jnp.bfloat16),
                preferred_element_type=jnp.float32) + b2e_ref[...]
    )
    W1n = W1n_ref[...].astype(jnp.bfloat16)
    hn = (
        jnp.dot(src, W1n[:DN], preferred_element_type=jnp.float32)
        + jnp.dot(ea, W1n[DN:], preferred_element_type=jnp.float32)
        + b1n_ref[...]
    )
    hn = jnp.maximum(hn, 0.0).astype(jnp.bfloat16)
    r = jnp.dot(hn, W2n_ref[...].astype(jnp.bfloat16),
                preferred_element_type=jnp.float32) + b2n_ref[...]
    bk = r.shape[0]
    lanes = v0_ref.shape[1]
    v0_ref[...] = r[:, :lanes]
    pad = 2 * lanes - r.shape[1] - 1
    v1_ref[...] = jnp.concatenate(
        [r[:, lanes:], jnp.ones((bk, 1), jnp.float32),
         jnp.zeros((bk, pad), jnp.float32)],
        axis=1,
    )


def _node_body(x_ref, *args):
    S_refs = args[:-5]
    W1m_ref, b1m_ref, W2m_ref, b2m_ref, node_ref = args[-5:]
    x = x_ref[...]
    DN = x.shape[1]
    DM = W1m_ref.shape[0] - DN
    lanes = S_refs[0].shape[2]
    S0 = sum(Sr[0] for Sr in S_refs)
    S1 = sum(Sr[1] for Sr in S_refs)
    summed = jnp.concatenate([S0, S1[:, :DM - lanes]], axis=1)
    cnt = S1[:, DM - lanes:DM - lanes + 1]
    mean = summed / jnp.maximum(cnt, 1.0)
    W1m = W1m_ref[...]
    h = (
        jnp.dot(x, W1m[:DN], preferred_element_type=jnp.float32)
        + jnp.dot(mean, W1m[DN:], preferred_element_type=jnp.float32)
        + b1m_ref[...]
    )
    h = jnp.maximum(h, 0.0)
    node_ref[...] = (
        jnp.dot(h, W2m_ref[...], preferred_element_type=jnp.float32) + b2m_ref[...]
    )


def kernel(x, edge_index, edge_attr,
           W1e, b1e, W2e, b2e,
           W1n, b1n, W2n, b2n,
           W1m, b1m, W2m, b2m):
    N, DN = x.shape
    E = edge_index.shape[1]
    DE = edge_attr.shape[1]
    DM = W2n.shape[1]
    LANES = 128
    WC = DM - LANES
    NL = W2m.shape[1]
    EH = E // SPLIT

    full = lambda shape: pl.BlockSpec(shape, lambda i: (0,) * len(shape))
    npad = ((N + NS * 8 - 1) // (NS * 8)) * (NS * 8)
    zeros_tile = jnp.zeros((npad // NS, LANES), jnp.float32)

    gather_h = _make_gather(EH, N, DN, jnp.float32)
    scatter_h = _make_scatter(EH, N, LANES, WC)

    def edge_tc(srch, desth, eah):
        BK = 2000
        return pl.pallas_call(
            _edge_body,
            grid=(EH // BK,),
            in_specs=[
                pl.BlockSpec((BK, DN), lambda i: (i, 0)),
                pl.BlockSpec((BK, DN), lambda i: (i, 0)),
                pl.BlockSpec((BK, DE), lambda i: (i, 0)),
                full(W1e.shape), full((1, b1e.shape[0])),
                full(W2e.shape), full((1, b2e.shape[0])),
                full(W1n.shape), full((1, b1n.shape[0])),
                full(W2n.shape), full((1, b2n.shape[0])),
            ],
            out_specs=[
                pl.BlockSpec((BK, W2e.shape[1]), lambda i: (i, 0)),
                pl.BlockSpec((BK, LANES), lambda i: (i, 0)),
                pl.BlockSpec((BK, LANES), lambda i: (i, 0)),
            ],
            out_shape=[
                jax.ShapeDtypeStruct((EH, W2e.shape[1]), jnp.float32),
                jax.ShapeDtypeStruct((EH, LANES), jnp.float32),
                jax.ShapeDtypeStruct((EH, LANES), jnp.float32),
            ],
        )(srch, desth, eah,
          W1e, b1e.reshape(1, -1), W2e, b2e.reshape(1, -1),
          W1n, b1n.reshape(1, -1), W2n, b2n.reshape(1, -1))

    edges = []
    S = []
    for h in range(SPLIT):
        rowh = lax.slice_in_dim(edge_index[0], h * EH, (h + 1) * EH)
        colh = lax.slice_in_dim(edge_index[1], h * EH, (h + 1) * EH)
        eah = lax.slice_in_dim(edge_attr, h * EH, (h + 1) * EH)
        srch, desth = gather_h(x, rowh, colh)
        edge_h, v0_h, v1_h = edge_tc(srch, desth, eah)
        S.append(scatter_h(v0_h, v1_h, colh, zeros_tile))
        edges.append(edge_h)

    edge_out = jnp.concatenate(edges, axis=0)

    BN = 2000
    node = pl.pallas_call(
        _node_body,
        grid=(N // BN,),
        in_specs=[
            pl.BlockSpec((BN, DN), lambda i: (i, 0)),
        ] + [
            pl.BlockSpec((NC, BN, LANES), lambda i: (0, i, 0))
            for _ in range(SPLIT)
        ] + [
            full(W1m.shape), full((1, b1m.shape[0])),
            full(W2m.shape), full((1, b2m.shape[0])),
        ],
        out_specs=pl.BlockSpec((BN, NL), lambda i: (i, 0)),
        out_shape=jax.ShapeDtypeStruct((N, NL), jnp.float32),
    )(x, *S, W1m, b1m.reshape(1, -1), W2m, b2m.reshape(1, -1))

    return (node, edge_out)


# speedup vs baseline: 1.2946x; 1.0144x over previous
import functools

import jax
import jax.numpy as jnp
from jax import lax
from jax.experimental import pallas as pl
from jax.experimental.pallas import tpu as pltpu
from jax.experimental.pallas import tpu_sc as plsc

NC = 2
NS = 16
NW = NC * NS
SPLIT = 5


def _pick_ch(epw):
    for c in range(128, 7, -8):
        if epw % c == 0:
            return c
    raise ValueError(epw)


def _sc_mesh():
    return plsc.VectorSubcoreMesh(
        core_axis_name="c", subcore_axis_name="s", num_cores=NC, num_subcores=NS
    )


def _pipeline(nchunk, proc, pref):
    npairs = (nchunk - 2) // 2
    pref(0, 0)
    pref(1, 1)

    def body(j2, carry):
        for b in range(2):
            jj = 2 * j2 + b
            proc(jj, b)
            pref(jj + 2, b)
        return carry

    lax.fori_loop(0, npairs, body, 0)
    done = 2 * npairs
    if nchunk % 2 == 1:
        proc(done, done % 2)
        pref(nchunk - 1, (nchunk - 1) % 2)
        done += 1
    for jj in range(done, nchunk):
        proc(jj, jj % 2)


def _make_gather(E, N, DN, dtype):
    epw = E // NW
    ch = _pick_ch(epw)
    nchunk = epw // ch

    @functools.partial(
        pl.kernel,
        out_type=(
            jax.ShapeDtypeStruct((E, DN), dtype),
            jax.ShapeDtypeStruct((E, DN), dtype),
        ),
        mesh=_sc_mesh(),
        scratch_types=[
            pltpu.VMEM((epw,), jnp.int32),
            pltpu.VMEM((epw,), jnp.int32),
            pltpu.VMEM((2, ch, DN), dtype),
            pltpu.VMEM((2, ch, DN), dtype),
            pltpu.SemaphoreType.DMA((2,)),
            pltpu.SemaphoreType.DMA((2,)),
        ],
    )
    def gather_k(x_hbm, row_hbm, col_hbm, src_hbm, dest_hbm,
                 idx_r, idx_c, buf_r, buf_c, sem_r, sem_c):
        wid = lax.axis_index("s") * NC + lax.axis_index("c")
        base0 = wid * epw
        pltpu.sync_copy(row_hbm.at[pl.ds(base0, epw)], idx_r)
        pltpu.sync_copy(col_hbm.at[pl.ds(base0, epw)], idx_c)

        def pref(jj, b):
            pltpu.async_copy(
                x_hbm.at[idx_r.at[pl.ds(jj * ch, ch)]], buf_r.at[b], sem_r.at[b])
            pltpu.async_copy(
                x_hbm.at[idx_c.at[pl.ds(jj * ch, ch)]], buf_c.at[b], sem_c.at[b])

        def proc(jj, b):
            base = base0 + jj * ch
            pltpu.make_async_copy(
                x_hbm.at[idx_r.at[pl.ds(jj * ch, ch)]], buf_r.at[b], sem_r.at[b]
            ).wait()
            pltpu.make_async_copy(
                x_hbm.at[idx_c.at[pl.ds(jj * ch, ch)]], buf_c.at[b], sem_c.at[b]
            ).wait()
            pltpu.sync_copy(buf_r.at[b], src_hbm.at[pl.ds(base, ch)])
            pltpu.sync_copy(buf_c.at[b], dest_hbm.at[pl.ds(base, ch)])

        _pipeline(nchunk, proc, pref)

    return gather_k


def _make_scatter(E, N, W, WC):
    epw = E // NS
    ch = _pick_ch(epw)
    nchunk = epw // ch
    npad = ((N + NS * 8 - 1) // (NS * 8)) * (NS * 8)
    rows_per_sub = npad // NS

    @functools.partial(
        pl.kernel,
        out_type=jax.ShapeDtypeStruct((NC, npad, W), jnp.float32),
        mesh=_sc_mesh(),
        scratch_types=[
            pltpu.VMEM((epw,), jnp.int32),
            pltpu.VMEM((2, ch, W), jnp.float32),
            pltpu.VMEM_SHARED((npad, W), jnp.float32),
            pltpu.SemaphoreType.DMA((2,)),
        ],
    )
    def scatter_k(v0_hbm, v1_hbm, col_hbm, z_hbm, out_hbm,
                  idx_v, val_v, acc_sh, sem):
        c = lax.axis_index("c")
        s = lax.axis_index("s")
        pltpu.sync_copy(z_hbm, acc_sh.at[pl.ds(s * rows_per_sub, rows_per_sub)])
        base0 = s * epw
        pltpu.sync_copy(col_hbm.at[pl.ds(base0, epw)], idx_v)
        plsc.subcore_barrier()

        def scat(jj, b):
            pltpu.sync_copy(
                val_v.at[b], acc_sh.at[idx_v.at[pl.ds(jj * ch, ch)]], add=True)

        def make(v_hbm):
            def pref(jj, b):
                pltpu.async_copy(
                    v_hbm.at[pl.ds(base0 + jj * ch, ch)], val_v.at[b], sem.at[b])

            def proc(jj, b):
                pltpu.make_async_copy(
                    v_hbm.at[pl.ds(base0, ch)], val_v.at[b], sem.at[b]).wait()
                scat(jj, b)

            return proc, pref

        @pl.when(c == 0)
        def _():
            _pipeline(nchunk, *make(v0_hbm))

        @pl.when(c == 1)
        def _():
            _pipeline(nchunk, *make(v1_hbm))

        plsc.subcore_barrier()
        pltpu.sync_copy(
            acc_sh.at[pl.ds(s * rows_per_sub, rows_per_sub)],
            out_hbm.at[c, pl.ds(s * rows_per_sub, rows_per_sub)],
        )

    return scatter_k


def _edge_body(src_ref, dest_ref, ea_ref, W1e_ref, b1e_ref, W2e_ref, b2e_ref,
               W1n_ref, b1n_ref, W2n_ref, b2n_ref, edge_ref, v0_ref, v1_ref):
    src = src_ref[...].astype(jnp.bfloat16)
    dest = dest_ref[...].astype(jnp.bfloat16)
    ea = ea_ref[...].astype(jnp.bfloat16)
    DN = src.shape[1]
    sd = jnp.concatenate([src, dest], axis=1)
    W1e = W1e_ref[...].astype(jnp.bfloat16)
    he = (
        jnp.dot(sd, W1e[:2 * DN], preferred_element_type=jnp.float32)
        + jnp.dot(ea, W1e[2 * DN:], preferred_element_type=jnp.float32)
        + b1e_ref[...]
    )
    he = jnp.maximum(he, 0.0).astype(jnp.bfloat16)
    edge_ref[...] = (
        jnp.dot(he, W2e_ref[...].astype(jnp.bfloat16),
                preferred_element_type=jnp.float32) + b2e_ref[...]
    )
    W1n = W1n_ref[...].astype(jnp.bfloat16)
    hn = (
        jnp.dot(jnp.concatenate([src, ea], axis=1), W1n,
                preferred_element_type=jnp.float32)
        + b1n_ref[...]
    )
    hn = jnp.maximum(hn, 0.0).astype(jnp.bfloat16)
    r = jnp.dot(hn, W2n_ref[...].astype(jnp.bfloat16),
                preferred_element_type=jnp.float32) + b2n_ref[...]
    bk = r.shape[0]
    lanes = v0_ref.shape[1]
    v0_ref[...] = r[:, :lanes]
    pad = 2 * lanes - r.shape[1] - 1
    v1_ref[...] = jnp.concatenate(
        [r[:, lanes:], jnp.ones((bk, 1), jnp.float32),
         jnp.zeros((bk, pad), jnp.float32)],
        axis=1,
    )


def _node_body(x_ref, *args):
    S_refs = args[:-5]
    W1m_ref, b1m_ref, W2m_ref, b2m_ref, node_ref = args[-5:]
    x = x_ref[...]
    DN = x.shape[1]
    DM = W1m_ref.shape[0] - DN
    lanes = S_refs[0].shape[2]
    S0 = sum(Sr[0] for Sr in S_refs)
    S1 = sum(Sr[1] for Sr in S_refs)
    summed = jnp.concatenate([S0, S1[:, :DM - lanes]], axis=1)
    cnt = S1[:, DM - lanes:DM - lanes + 1]
    mean = summed / jnp.maximum(cnt, 1.0)
    W1m = W1m_ref[...]
    h = (
        jnp.dot(x, W1m[:DN], preferred_element_type=jnp.float32)
        + jnp.dot(mean, W1m[DN:], preferred_element_type=jnp.float32)
        + b1m_ref[...]
    )
    h = jnp.maximum(h, 0.0)
    node_ref[...] = (
        jnp.dot(h, W2m_ref[...], preferred_element_type=jnp.float32) + b2m_ref[...]
    )


def kernel(x, edge_index, edge_attr,
           W1e, b1e, W2e, b2e,
           W1n, b1n, W2n, b2n,
           W1m, b1m, W2m, b2m):
    N, DN = x.shape
    E = edge_index.shape[1]
    DE = edge_attr.shape[1]
    DM = W2n.shape[1]
    LANES = 128
    WC = DM - LANES
    NL = W2m.shape[1]
    EH = E // SPLIT

    full = lambda shape: pl.BlockSpec(shape, lambda i: (0,) * len(shape))
    npad = ((N + NS * 8 - 1) // (NS * 8)) * (NS * 8)
    zeros_tile = jnp.zeros((npad // NS, LANES), jnp.float32)

    gather_h = _make_gather(EH, N, DN, jnp.float32)
    scatter_h = _make_scatter(EH, N, LANES, WC)

    def edge_tc(srch, desth, eah):
        BK = 2000
        return pl.pallas_call(
            _edge_body,
            grid=(EH // BK,),
            in_specs=[
                pl.BlockSpec((BK, DN), lambda i: (i, 0)),
                pl.BlockSpec((BK, DN), lambda i: (i, 0)),
                pl.BlockSpec((BK, DE), lambda i: (i, 0)),
                full(W1e.shape), full((1, b1e.shape[0])),
                full(W2e.shape), full((1, b2e.shape[0])),
                full(W1n.shape), full((1, b1n.shape[0])),
                full(W2n.shape), full((1, b2n.shape[0])),
            ],
            out_specs=[
                pl.BlockSpec((BK, W2e.shape[1]), lambda i: (i, 0)),
                pl.BlockSpec((BK, LANES), lambda i: (i, 0)),
                pl.BlockSpec((BK, LANES), lambda i: (i, 0)),
            ],
            out_shape=[
                jax.ShapeDtypeStruct((EH, W2e.shape[1]), jnp.float32),
                jax.ShapeDtypeStruct((EH, LANES), jnp.float32),
                jax.ShapeDtypeStruct((EH, LANES), jnp.float32),
            ],
        )(srch, desth, eah,
          W1e, b1e.reshape(1, -1), W2e, b2e.reshape(1, -1),
          W1n, b1n.reshape(1, -1), W2n, b2n.reshape(1, -1))

    edges = []
    S = []
    for h in range(SPLIT):
        rowh = lax.slice_in_dim(edge_index[0], h * EH, (h + 1) * EH)
        colh = lax.slice_in_dim(edge_index[1], h * EH, (h + 1) * EH)
        eah = lax.slice_in_dim(edge_attr, h * EH, (h + 1) * EH)
        srch, desth = gather_h(x, rowh, colh)
        edge_h, v0_h, v1_h = edge_tc(srch, desth, eah)
        S.append(scatter_h(v0_h, v1_h, colh, zeros_tile))
        edges.append(edge_h)

    edge_out = jnp.concatenate(edges, axis=0)

    BN = 2000
    node = pl.pallas_call(
        _node_body,
        grid=(N // BN,),
        in_specs=[
            pl.BlockSpec((BN, DN), lambda i: (i, 0)),
        ] + [
            pl.BlockSpec((NC, BN, LANES), lambda i: (0, i, 0))
            for _ in range(SPLIT)
        ] + [
            full(W1m.shape), full((1, b1m.shape[0])),
            full(W2m.shape), full((1, b2m.shape[0])),
        ],
        out_specs=pl.BlockSpec((BN, NL), lambda i: (i, 0)),
        out_shape=jax.ShapeDtypeStruct((N, NL), jnp.float32),
    )(x, *S, W1m, b1m.reshape(1, -1), W2m, b2m.reshape(1, -1))

    return (node, edge_out)


# speedup vs baseline: 1.2953x; 1.0005x over previous
import functools

import jax
import jax.numpy as jnp
from jax import lax
from jax.experimental import pallas as pl
from jax.experimental.pallas import tpu as pltpu
from jax.experimental.pallas import tpu_sc as plsc

NC = 2
NS = 16
NW = NC * NS
SPLIT = 5


def _pick_ch(epw):
    for c in range(128, 7, -8):
        if epw % c == 0:
            return c
    raise ValueError(epw)


def _sc_mesh():
    return plsc.VectorSubcoreMesh(
        core_axis_name="c", subcore_axis_name="s", num_cores=NC, num_subcores=NS
    )


def _pipeline(nchunk, proc, pref):
    npairs = (nchunk - 2) // 2
    pref(0, 0)
    pref(1, 1)

    def body(j2, carry):
        for b in range(2):
            jj = 2 * j2 + b
            proc(jj, b)
            pref(jj + 2, b)
        return carry

    lax.fori_loop(0, npairs, body, 0)
    done = 2 * npairs
    if nchunk % 2 == 1:
        proc(done, done % 2)
        pref(nchunk - 1, (nchunk - 1) % 2)
        done += 1
    for jj in range(done, nchunk):
        proc(jj, jj % 2)


def _make_gather(E, N, DN, dtype):
    epw = E // NW
    ch = _pick_ch(epw)
    nchunk = epw // ch

    @functools.partial(
        pl.kernel,
        out_type=(
            jax.ShapeDtypeStruct((E, DN), dtype),
            jax.ShapeDtypeStruct((E, DN), dtype),
        ),
        mesh=_sc_mesh(),
        scratch_types=[
            pltpu.VMEM((epw,), jnp.int32),
            pltpu.VMEM((epw,), jnp.int32),
            pltpu.VMEM((2, ch, DN), dtype),
            pltpu.VMEM((2, ch, DN), dtype),
            pltpu.SemaphoreType.DMA((2,)),
            pltpu.SemaphoreType.DMA((2,)),
        ],
    )
    def gather_k(x_hbm, row_hbm, col_hbm, src_hbm, dest_hbm,
                 idx_r, idx_c, buf_r, buf_c, sem_r, sem_c):
        wid = lax.axis_index("s") * NC + lax.axis_index("c")
        base0 = wid * epw
        pltpu.sync_copy(row_hbm.at[pl.ds(base0, epw)], idx_r)
        pltpu.sync_copy(col_hbm.at[pl.ds(base0, epw)], idx_c)

        def pref(jj, b):
            pltpu.async_copy(
                x_hbm.at[idx_r.at[pl.ds(jj * ch, ch)]], buf_r.at[b], sem_r.at[b])
            pltpu.async_copy(
                x_hbm.at[idx_c.at[pl.ds(jj * ch, ch)]], buf_c.at[b], sem_c.at[b])

        def proc(jj, b):
            base = base0 + jj * ch
            pltpu.make_async_copy(
                x_hbm.at[idx_r.at[pl.ds(jj * ch, ch)]], buf_r.at[b], sem_r.at[b]
            ).wait()
            pltpu.make_async_copy(
                x_hbm.at[idx_c.at[pl.ds(jj * ch, ch)]], buf_c.at[b], sem_c.at[b]
            ).wait()
            pltpu.sync_copy(buf_r.at[b], src_hbm.at[pl.ds(base, ch)])
            pltpu.sync_copy(buf_c.at[b], dest_hbm.at[pl.ds(base, ch)])

        _pipeline(nchunk, proc, pref)

    return gather_k


def _make_scatter(E, N, W):
    epw = E // NS
    ch = _pick_ch(epw)
    nchunk = epw // ch
    npad = ((N + NS * 8 - 1) // (NS * 8)) * (NS * 8)
    rows_per_sub = npad // NS

    @functools.partial(
        pl.kernel,
        out_type=jax.ShapeDtypeStruct((NC, npad, W), jnp.float32),
        mesh=_sc_mesh(),
        scratch_types=[
            pltpu.VMEM((epw,), jnp.int32),
            pltpu.VMEM((2, ch, W), jnp.float32),
            pltpu.VMEM_SHARED((npad, W), jnp.float32),
            pltpu.SemaphoreType.DMA((2,)),
        ],
    )
    def scatter_k(v0_hbm, v1_hbm, col_hbm, z_hbm, out_hbm,
                  idx_v, val_v, acc_sh, sem):
        c = lax.axis_index("c")
        s = lax.axis_index("s")
        pltpu.sync_copy(z_hbm, acc_sh.at[pl.ds(s * rows_per_sub, rows_per_sub)])
        base0 = s * epw
        pltpu.sync_copy(col_hbm.at[pl.ds(base0, epw)], idx_v)
        plsc.subcore_barrier()

        def scat(jj, b):
            pltpu.sync_copy(
                val_v.at[b], acc_sh.at[idx_v.at[pl.ds(jj * ch, ch)]], add=True)

        def make(v_hbm):
            def pref(jj, b):
                pltpu.async_copy(
                    v_hbm.at[pl.ds(base0 + jj * ch, ch)], val_v.at[b], sem.at[b])

            def proc(jj, b):
                pltpu.make_async_copy(
                    v_hbm.at[pl.ds(base0, ch)], val_v.at[b], sem.at[b]).wait()
                scat(jj, b)

            return proc, pref

        @pl.when(c == 0)
        def _():
            _pipeline(nchunk, *make(v0_hbm))

        @pl.when(c == 1)
        def _():
            _pipeline(nchunk, *make(v1_hbm))

        plsc.subcore_barrier()
        pltpu.sync_copy(
            acc_sh.at[pl.ds(s * rows_per_sub, rows_per_sub)],
            out_hbm.at[c, pl.ds(s * rows_per_sub, rows_per_sub)],
        )

    return scatter_k


def _edge_body(src_ref, dest_ref, ea_ref, W1e_ref, b1e_ref, W2e_ref, b2e_ref,
               W1n_ref, b1n_ref, W2n_ref, b2n_ref, edge_ref, v0_ref, v1_ref):
    src = src_ref[...].astype(jnp.bfloat16)
    dest = dest_ref[...].astype(jnp.bfloat16)
    ea = ea_ref[...].astype(jnp.bfloat16)
    DN = src.shape[1]
    sd = jnp.concatenate([src, dest], axis=1)
    W1e = W1e_ref[...].astype(jnp.bfloat16)
    he = (
        jnp.dot(sd, W1e[:2 * DN], preferred_element_type=jnp.float32)
        + jnp.dot(ea, W1e[2 * DN:], preferred_element_type=jnp.float32)
        + b1e_ref[...]
    )
    he = jnp.maximum(he, 0.0).astype(jnp.bfloat16)
    edge_ref[...] = (
        jnp.dot(he, W2e_ref[...].astype(jnp.bfloat16),
                preferred_element_type=jnp.float32) + b2e_ref[...]
    )
    W1n = W1n_ref[...].astype(jnp.bfloat16)
    hn = (
        jnp.dot(jnp.concatenate([src, ea], axis=1), W1n,
                preferred_element_type=jnp.float32)
        + b1n_ref[...]
    )
    hn = jnp.maximum(hn, 0.0).astype(jnp.bfloat16)
    r = jnp.dot(hn, W2n_ref[...].astype(jnp.bfloat16),
                preferred_element_type=jnp.float32) + b2n_ref[...]
    bk = r.shape[0]
    lanes = v0_ref.shape[1]
    v0_ref[...] = r[:, :lanes]
    pad = 2 * lanes - r.shape[1] - 1
    v1_ref[...] = jnp.concatenate(
        [r[:, lanes:], jnp.ones((bk, 1), jnp.float32),
         jnp.zeros((bk, pad), jnp.float32)],
        axis=1,
    )


def _node_body(x_ref, *args):
    S_refs = args[:-5]
    W1m_ref, b1m_ref, W2m_ref, b2m_ref, node_ref = args[-5:]
    x = x_ref[...]
    DN = x.shape[1]
    DM = W1m_ref.shape[0] - DN
    lanes = S_refs[0].shape[2]
    S0 = sum(Sr[0] for Sr in S_refs)
    S1 = sum(Sr[1] for Sr in S_refs)
    summed = jnp.concatenate([S0, S1[:, :DM - lanes]], axis=1)
    cnt = S1[:, DM - lanes:DM - lanes + 1]
    mean = summed / jnp.maximum(cnt, 1.0)
    W1m = W1m_ref[...]
    h = (
        jnp.dot(x, W1m[:DN], preferred_element_type=jnp.float32)
        + jnp.dot(mean, W1m[DN:], preferred_element_type=jnp.float32)
        + b1m_ref[...]
    )
    h = jnp.maximum(h, 0.0)
    node_ref[...] = (
        jnp.dot(h, W2m_ref[...], preferred_element_type=jnp.float32) + b2m_ref[...]
    )


def kernel(x, edge_index, edge_attr,
           W1e, b1e, W2e, b2e,
           W1n, b1n, W2n, b2n,
           W1m, b1m, W2m, b2m):
    N, DN = x.shape
    E = edge_index.shape[1]
    DE = edge_attr.shape[1]
    DM = W2n.shape[1]
    LANES = 128
    NL = W2m.shape[1]
    EH = E // SPLIT

    full = lambda shape: pl.BlockSpec(shape, lambda i: (0,) * len(shape))
    npad = ((N + NS * 8 - 1) // (NS * 8)) * (NS * 8)
    zeros_tile = jnp.zeros((npad // NS, LANES), jnp.float32)

    gather_h = _make_gather(EH, N, DN, jnp.float32)
    scatter_h = _make_scatter(EH, N, LANES)

    def edge_tc(srch, desth, eah):
        BK = 2000
        return pl.pallas_call(
            _edge_body,
            grid=(EH // BK,),
            in_specs=[
                pl.BlockSpec((BK, DN), lambda i: (i, 0)),
                pl.BlockSpec((BK, DN), lambda i: (i, 0)),
                pl.BlockSpec((BK, DE), lambda i: (i, 0)),
                full(W1e.shape), full((1, b1e.shape[0])),
                full(W2e.shape), full((1, b2e.shape[0])),
                full(W1n.shape), full((1, b1n.shape[0])),
                full(W2n.shape), full((1, b2n.shape[0])),
            ],
            out_specs=[
                pl.BlockSpec((BK, W2e.shape[1]), lambda i: (i, 0)),
                pl.BlockSpec((BK, LANES), lambda i: (i, 0)),
                pl.BlockSpec((BK, LANES), lambda i: (i, 0)),
            ],
            out_shape=[
                jax.ShapeDtypeStruct((EH, W2e.shape[1]), jnp.float32),
                jax.ShapeDtypeStruct((EH, LANES), jnp.float32),
                jax.ShapeDtypeStruct((EH, LANES), jnp.float32),
            ],
        )(srch, desth, eah,
          W1e, b1e.reshape(1, -1), W2e, b2e.reshape(1, -1),
          W1n, b1n.reshape(1, -1), W2n, b2n.reshape(1, -1))

    edges = []
    S = []
    for h in range(SPLIT):
        rowh = lax.slice_in_dim(edge_index[0], h * EH, (h + 1) * EH)
        colh = lax.slice_in_dim(edge_index[1], h * EH, (h + 1) * EH)
        eah = lax.slice_in_dim(edge_attr, h * EH, (h + 1) * EH)
        srch, desth = gather_h(x, rowh, colh)
        edge_h, v0_h, v1_h = edge_tc(srch, desth, eah)
        S.append(scatter_h(v0_h, v1_h, colh, zeros_tile))
        edges.append(edge_h)

    edge_out = jnp.concatenate(edges, axis=0)

    BN = 2000
    node = pl.pallas_call(
        _node_body,
        grid=(N // BN,),
        in_specs=[
            pl.BlockSpec((BN, DN), lambda i: (i, 0)),
        ] + [
            pl.BlockSpec((NC, BN, LANES), lambda i: (0, i, 0))
            for _ in range(SPLIT)
        ] + [
            full(W1m.shape), full((1, b1m.shape[0])),
            full(W2m.shape), full((1, b2m.shape[0])),
        ],
        out_specs=pl.BlockSpec((BN, NL), lambda i: (i, 0)),
        out_shape=jax.ShapeDtypeStruct((N, NL), jnp.float32),
    )(x, *S, W1m, b1m.reshape(1, -1), W2m, b2m.reshape(1, -1))

    return (node, edge_out)
